# baseline (speedup 1.0000x reference)
import functools

import jax
import jax.numpy as jnp
from jax import lax
from jax.experimental import pallas as pl
from jax.experimental.pallas import tpu as pltpu
from jax.experimental.pallas import tpu_sc as plsc

_NUM_CORES = 2
_NUM_SUBCORES = 16
_NUM_WORKERS = _NUM_CORES * _NUM_SUBCORES
_CHUNK = 128
_NSLOT = 8
_AHEAD = 6


def kernel(token_ids, weight):
    batch, seq = token_ids.shape
    dim = weight.shape[1]
    total = batch * seq

    per_worker = total // _NUM_WORKERS
    chunks = per_worker // _CHUNK

    mesh = plsc.VectorSubcoreMesh(core_axis_name="c", subcore_axis_name="s")

    @functools.partial(
        pl.kernel,
        mesh=mesh,
        out_type=jax.ShapeDtypeStruct((total, dim), weight.dtype),
        scratch_types=[
            pltpu.VMEM((per_worker,), jnp.int32),
            pltpu.VMEM((_NSLOT, _CHUNK, dim), jnp.float32),
            pltpu.SemaphoreType.DMA((_NSLOT,)),
            pltpu.SemaphoreType.DMA((_NSLOT,)),
        ],
        compiler_params=pltpu.CompilerParams(use_tc_tiling_on_sc=False),
    )
    def gather_kernel(table_hbm, idx_hbm, out_hbm, idx_v, rows_v, gsem, osem):
        wid = lax.axis_index("s") * _NUM_CORES + lax.axis_index("c")
        base = wid * per_worker
        pltpu.sync_copy(idx_hbm.at[pl.ds(base, per_worker)], idx_v)

        gather_d = {}
        out_d = {}

        def start_gather(c):
            slot = c % _NSLOT
            gather_d[c] = pltpu.async_copy(
                table_hbm.at[idx_v.at[pl.ds(c * _CHUNK, _CHUNK)]],
                rows_v.at[slot],
                gsem.at[slot],
            )

        def start_out(c):
            slot = c % _NSLOT
            out_d[c] = pltpu.async_copy(
                rows_v.at[slot],
                out_hbm.at[pl.ds(base + c * _CHUNK, _CHUNK)],
                osem.at[slot],
            )

        for j in range(_AHEAD):
            start_gather(j)
        for c in range(chunks):
            j = c + _AHEAD
            if j < chunks:
                if j >= _NSLOT:
                    out_d[j - _NSLOT].wait()
                start_gather(j)
            gather_d[c].wait()
            start_out(c)
        for c in range(max(0, chunks - _NSLOT), chunks):
            out_d[c].wait()

    flat_ids = token_ids.astype(jnp.int32).reshape(total)
    out = gather_kernel(weight, flat_ids)
    return out.reshape(batch, seq, dim)
